# Initial kernel scaffold; baseline (speedup 1.0000x reference)
import jax
import jax.numpy as jnp
from jax.experimental import pallas as pl


def kernel(x, edge_index, edge_attr, batch, Wq, Wk, Wv, We, Wroot, b):
    raise NotImplementedError("write your pallas kernel here")


# speedup vs baseline: 2.4055x; 2.4055x over previous
import jax
import jax.numpy as jnp
from jax import lax
from jax.experimental import pallas as pl
from jax.experimental.pallas import tpu as pltpu
from jax.experimental.pallas import tpu_sc as plsc

N = 10000
E = 320000
D = 128
DE = 16
G = 64

NC = 2
NS = 16
NW = NC * NS
C = 128
NCHUNK = 79
EPW = NCHUNK * C
E_PAD = NW * EPW
N_PAD = 10112
RPT = N_PAD // NS
ZROW = N

EB = 10112
ESTEPS = E_PAD // EB


def _prep_body(x_ref, wq_ref, wk_ref, wv_ref, wr_ref, we_ref, b_ref,
               eattr_ref, q_ref, k_ref, v_ref, xr_ref, ea_ref):
    @pl.when(pl.program_id(0) == 0)
    def _():
        xv = x_ref[...]
        q_ref[...] = jnp.dot(xv, wq_ref[...], preferred_element_type=jnp.float32)
        k_ref[...] = jnp.dot(xv, wk_ref[...], preferred_element_type=jnp.float32)
        v_ref[...] = jnp.dot(xv, wv_ref[...], preferred_element_type=jnp.float32)
        xr_ref[...] = (
            jnp.dot(xv, wr_ref[...], preferred_element_type=jnp.float32)
            + b_ref[...]
        )

    ea_ref[...] = jnp.dot(eattr_ref[...], we_ref[...],
                          preferred_element_type=jnp.float32)


_prep = pl.pallas_call(
    _prep_body,
    grid=(ESTEPS,),
    in_specs=[
        pl.BlockSpec((N_PAD, D), lambda i: (0, 0)),
        pl.BlockSpec((D, D), lambda i: (0, 0)),
        pl.BlockSpec((D, D), lambda i: (0, 0)),
        pl.BlockSpec((D, D), lambda i: (0, 0)),
        pl.BlockSpec((D, D), lambda i: (0, 0)),
        pl.BlockSpec((DE, D), lambda i: (0, 0)),
        pl.BlockSpec((1, D), lambda i: (0, 0)),
        pl.BlockSpec((EB, DE), lambda i: (i, 0)),
    ],
    out_specs=[
        pl.BlockSpec((N_PAD, D), lambda i: (0, 0)),
        pl.BlockSpec((N_PAD, D), lambda i: (0, 0)),
        pl.BlockSpec((N_PAD, D), lambda i: (0, 0)),
        pl.BlockSpec((N_PAD, D), lambda i: (0, 0)),
        pl.BlockSpec((EB, D), lambda i: (i, 0)),
    ],
    out_shape=[
        jax.ShapeDtypeStruct((N_PAD, D), jnp.float32),
        jax.ShapeDtypeStruct((N_PAD, D), jnp.float32),
        jax.ShapeDtypeStruct((N_PAD, D), jnp.float32),
        jax.ShapeDtypeStruct((N_PAD, D), jnp.float32),
        jax.ShapeDtypeStruct((E_PAD, D), jnp.float32),
    ],
)


def _edge_body(q_hbm, k_hbm, v_hbm, ea_hbm, dst_hbm, src_hbm, out_hbm,
               qa, kb, eab, dsti, srci, agg_sh, sem_a, sem_b):
    cid = lax.axis_index("c")
    sid = lax.axis_index("s")
    wid = cid * NS + sid

    def zrow(r, carry):
        for j in range(D // 16):
            qa[r, pl.ds(j * 16, 16)] = jnp.zeros((16,), jnp.float32)
        return carry

    lax.fori_loop(0, C, zrow, 0)
    for k in range(4):
        pltpu.sync_copy(qa, agg_sh.at[pl.ds(sid * RPT + k * C, C)])
    pltpu.sync_copy(qa.at[pl.ds(0, RPT - 4 * C)],
                    agg_sh.at[pl.ds(sid * RPT + 4 * C, RPT - 4 * C)])
    plsc.subcore_barrier()

    def chunk(c, carry):
        pltpu.sync_copy(dst_hbm.at[wid, c], dsti)
        pltpu.sync_copy(src_hbm.at[wid, c], srci)
        cp_q = pltpu.async_copy(q_hbm.at[dsti], qa, sem_a)
        cp_k = pltpu.async_copy(k_hbm.at[srci], kb, sem_b)
        pltpu.sync_copy(ea_hbm.at[pl.ds(wid * EPW + c * C, C)], eab)
        cp_q.wait()
        cp_k.wait()

        def zpass(r, rcarry):
            for j in range(D // 16):
                sl = pl.ds(j * 16, 16)
                eab[r, sl] = qa[r, sl] + kb[r, sl] + eab[r, sl]
            return rcarry

        lax.fori_loop(0, C, zpass, 0)
        pltpu.async_copy(v_hbm.at[srci], kb, sem_b).wait()

        def mpass(r, rcarry):
            for j in range(D // 16):
                sl = pl.ds(j * 16, 16)
                qa[r, sl] = kb[r, sl] / (1.0 + jnp.exp(-eab[r, sl]))
            return rcarry

        lax.fori_loop(0, C, mpass, 0)
        pltpu.sync_copy(qa, agg_sh.at[dsti], add=True)
        return carry

    lax.fori_loop(0, NCHUNK, chunk, 0)
    plsc.subcore_barrier()

    for k in range(4):
        off = sid * RPT + k * C
        pltpu.sync_copy(agg_sh.at[pl.ds(off, C)], qa)
        pltpu.sync_copy(qa, out_hbm.at[cid, pl.ds(off, C)])
    tail = RPT - 4 * C
    off = sid * RPT + 4 * C
    pltpu.sync_copy(agg_sh.at[pl.ds(off, tail)], qa.at[pl.ds(0, tail)])
    pltpu.sync_copy(qa.at[pl.ds(0, tail)], out_hbm.at[cid, pl.ds(off, tail)])


_edge = pl.kernel(
    _edge_body,
    out_type=jax.ShapeDtypeStruct((NC, N_PAD, D), jnp.float32),
    mesh=plsc.VectorSubcoreMesh(core_axis_name="c", subcore_axis_name="s"),
    scratch_types=[
        pltpu.VMEM((C, D), jnp.float32),
        pltpu.VMEM((C, D), jnp.float32),
        pltpu.VMEM((C, D), jnp.float32),
        pltpu.VMEM((C,), jnp.int32),
        pltpu.VMEM((C,), jnp.int32),
        pltpu.VMEM_SHARED((N_PAD, D), jnp.float32),
        pltpu.SemaphoreType.DMA,
        pltpu.SemaphoreType.DMA,
    ],
)


def _pool_body(xr_ref, a0_ref, a1_ref, batch_ref, out_ref):
    out = jnp.maximum(xr_ref[...] + a0_ref[...] + a1_ref[...], 0.0)
    sel = (batch_ref[...] ==
           lax.broadcasted_iota(jnp.int32, (N, G), 1)).astype(jnp.float32)
    psum = lax.dot_general(sel, out, (((0,), (0,)), ((), ())),
                           preferred_element_type=jnp.float32)
    cnts = lax.dot_general(sel, jnp.ones((N, D), jnp.float32),
                           (((0,), (0,)), ((), ())),
                           preferred_element_type=jnp.float32)
    out_ref[...] = psum / jnp.maximum(cnts, 1.0)


_pool = pl.pallas_call(
    _pool_body,
    out_shape=jax.ShapeDtypeStruct((G, D), jnp.float32),
)


def kernel(x, edge_index, edge_attr, batch, Wq, Wk, Wv, We, Wroot, b):
    pad_e = E_PAD - E
    src3 = jnp.concatenate(
        [edge_index[0], jnp.full((pad_e,), ZROW, jnp.int32)]
    ).reshape(NW, NCHUNK, C)
    dst3 = jnp.concatenate(
        [edge_index[1], jnp.zeros((pad_e,), jnp.int32)]
    ).reshape(NW, NCHUNK, C)
    x_pad = jnp.pad(x, ((0, N_PAD - N), (0, 0)))
    ea_pad = jnp.pad(edge_attr, ((0, pad_e), (0, 0)))
    q, k, v, xr, ea = _prep(x_pad, Wq, Wk, Wv, Wroot, We, b.reshape(1, D),
                            ea_pad)
    aggs = _edge(q, k, v, ea, dst3, src3)
    return _pool(xr[:N], aggs[0, :N], aggs[1, :N], batch.reshape(N, 1))
